# baseline (speedup 1.0000x reference)
import functools

import jax
import jax.numpy as jnp
from jax import lax
from jax.experimental import pallas as pl
from jax.experimental.pallas import tpu as pltpu
from jax.experimental.pallas import tpu_sc as plsc

EMB_DIM = 64
BATCH = 16384
NROWS = 1000001

NUM_CORES = 2
NUM_SUBCORES = 16
NUM_WORKERS = NUM_CORES * NUM_SUBCORES
LANES = 16

KCOLS = 7813
KW = 246
CK = 2
NCH = KW // CK
CLS = CK * 128
K0_MAX = (NROWS - CLS) // 128
TAIL_LO = 128 * (K0_MAX + CK)
TAIL_W = NROWS - TAIL_LO
W_CLS = KW * 128
IDX_ROWS = BATCH // LANES
TRASH = BATCH


@functools.partial(
    pl.kernel,
    mesh=plsc.VectorSubcoreMesh(core_axis_name="c", subcore_axis_name="s"),
    out_type=jax.ShapeDtypeStruct((BATCH + 1, 128), jnp.float32),
    compiler_params=pltpu.CompilerParams(needs_layout_passes=False),
    scratch_types=[
        pltpu.VMEM((BATCH,), jnp.int32),
        pltpu.VMEM((BATCH,), jnp.int32),
        pltpu.VMEM((BATCH,), jnp.int32),
        pltpu.VMEM((2, 8, 8, CLS), jnp.float32),
        pltpu.VMEM((8, 8, TAIL_W), jnp.float32),
        pltpu.VMEM((LANES, 128), jnp.float32),
        pltpu.SemaphoreType.DMA,
        pltpu.SemaphoreType.DMA,
        pltpu.SemaphoreType.DMA,
    ],
)
def _sc_stream_gather(idx_hbm, tt_hbm, out_hbm, idx_v, mpos_v, mcls_v,
                      buf_v, tail_v, stg_v, dsem0, dsem1, ssem):
    wid = lax.axis_index("s") * NUM_CORES + lax.axis_index("c")
    pltpu.sync_copy(idx_hbm, idx_v)

    iota = lax.iota(jnp.int32, LANES)
    lo = wid * W_CLS
    hi = jnp.where(wid == NUM_WORKERS - 1, NROWS, lo + W_CLS)

    def scan(t, n):
        v = plsc.load_gather(idx_v, [t * LANES + iota])
        m = (v >= lo) & (v < hi)
        c1 = plsc.cumsum(jnp.where(m, 1, 0))
        dst = n + c1 - 1
        plsc.store_scatter(mpos_v, [dst], t * LANES + iota, mask=m)
        plsc.store_scatter(mcls_v, [dst], v, mask=m)
        return n + c1[LANES - 1]

    n = lax.fori_loop(0, IDX_ROWS, scan, jnp.int32(0))
    pad = n + iota
    plsc.store_scatter(mcls_v, [pad], jnp.full((LANES,), -1, jnp.int32),
                       mask=pad < BATCH)
    nvr = (n + LANES - 1) >> 4

    def fire_chunk(c, par):
        k0 = pl.multiple_of(
            jnp.minimum(wid * KW + CK * c, K0_MAX) * 128, 128)
        sem = dsem0 if par == 0 else dsem1
        for a in range(8):
            pltpu.async_copy(
                tt_hbm.at[pl.ds(8 * a, 8), pl.ds(k0, CLS)],
                buf_v.at[par, a], sem)

    def wait_chunk(par):
        sem = dsem0 if par == 0 else dsem1
        for a in range(8):
            pltpu.make_async_copy(
                tt_hbm.at[pl.ds(0, 8), pl.ds(0, CLS)],
                buf_v.at[par, a], sem).wait()

    def wait_scatter():
        pltpu.make_async_copy(
            stg_v, out_hbm.at[pl.ds(0, LANES), :], ssem).wait()

    def assemble_and_scatter(src_ref, j, clo, width):
        mp = plsc.load_gather(mpos_v, [j * LANES + iota])
        mc = plsc.load_gather(mcls_v, [j * LANES + iota])
        inb = (mc >= clo) & (mc < clo + width)
        hits = plsc.all_reduce_population_count(inb)

        def do16():
            wait_scatter()
            moff = jnp.clip(mc - clo, 0, width - 1)
            for f in range(LANES):
                mo = jnp.full((LANES,), 1, jnp.int32) * moff[f]
                for k in range(EMB_DIM // LANES):
                    cv = iota + k * LANES
                    vals = plsc.load_gather(
                        src_ref, [cv >> 3, cv & 7, mo])
                    stg_v[f, pl.ds(k * LANES, LANES)] = vals
            pos = jnp.where(inb, mp, TRASH)
            pltpu.async_copy(stg_v, out_hbm.at[pos], ssem)

        pl.when(hits[0] > 0)(do16)

    def process_chunk(c, par, width):
        clo = jnp.minimum(wid * KW + CK * c, K0_MAX) * 128

        def pbody(j, _):
            assemble_and_scatter(buf_v.at[par], j, clo, width)
            return ()

        lax.fori_loop(0, nvr, pbody, ())

    pltpu.async_copy(stg_v, out_hbm.at[jnp.full((LANES,), TRASH, jnp.int32)],
                     ssem)
    fire_chunk(0, 0)

    def step(c, par):
        def go():
            pl.when(c < NCH - 1)(lambda: fire_chunk(c + 1, 1 - par))
            wait_chunk(par)
            process_chunk(c, par, CLS)

        return go

    def body(c, _):
        pl.when(c % 2 == 0)(step(c, 0))
        pl.when(c % 2 == 1)(step(c, 1))
        return ()

    lax.fori_loop(0, NCH, body, ())

    def tail():
        for a in range(8):
            pltpu.async_copy(
                tt_hbm.at[pl.ds(8 * a, 8), pl.ds(TAIL_LO, TAIL_W)],
                tail_v.at[a], dsem0)
        for a in range(8):
            pltpu.make_async_copy(
                tt_hbm.at[pl.ds(0, 8), pl.ds(TAIL_LO, TAIL_W)],
                tail_v.at[a], dsem0).wait()

        def pbody(j, _):
            assemble_and_scatter(tail_v, j, jnp.int32(TAIL_LO), TAIL_W)
            return ()

        lax.fori_loop(0, nvr, pbody, ())

    pl.when(wid == NUM_WORKERS - 1)(tail)
    wait_scatter()


def kernel(x, table):
    out2 = _sc_stream_gather(x.astype(jnp.int32), table.T)
    return out2[:BATCH, :EMB_DIM]
